# baseline (speedup 1.0000x reference)
import jax
import jax.numpy as jnp
from jax.experimental import pallas as pl
from jax.experimental.pallas import tpu as pltpu

B, H, S, D = 16, 8, 2048, 128
ROWS = B * H
NCHUNK = 8


def _dma_body(k_ref, v_ref, ko_ref, vo_ref, sems):
    n = ROWS // NCHUNK
    copies = []
    for c in range(NCHUNK):
        sl = pl.ds(c * n, n)
        copies.append(pltpu.make_async_copy(k_ref.at[sl], ko_ref.at[sl], sems.at[2 * c]))
        copies.append(pltpu.make_async_copy(v_ref.at[sl], vo_ref.at[sl], sems.at[2 * c + 1]))
    for cp in copies:
        cp.start()
    for cp in copies:
        cp.wait()


def kernel(k_val, v_val, k_cache, v_cache):
    k2 = k_val.reshape(ROWS, S, D)
    v2 = v_val.reshape(ROWS, S, D)
    out = pl.pallas_call(
        _dma_body,
        in_specs=[pl.BlockSpec(memory_space=pl.ANY)] * 2,
        out_specs=[pl.BlockSpec(memory_space=pl.ANY)] * 2,
        out_shape=[jax.ShapeDtypeStruct((ROWS, S, D), jnp.float32)] * 2,
        scratch_shapes=[pltpu.SemaphoreType.DMA((2 * NCHUNK,))],
    )(k2, v2)
    return out[0].reshape(B, H, S, D), out[1].reshape(B, H, S, D)


# speedup vs baseline: 48.0231x; 48.0231x over previous
import jax
import jax.numpy as jnp
from jax.experimental import pallas as pl

BR = 2


def _copy_body(k_ref, v_ref, ko_ref, vo_ref):
    ko_ref[...] = k_ref[...]
    vo_ref[...] = v_ref[...]


def kernel(k_val, v_val, k_cache, v_cache):
    B, H, S, D = k_val.shape
    rows = B * H
    k2 = k_val.reshape(rows, S, D)
    v2 = v_val.reshape(rows, S, D)
    spec = pl.BlockSpec((BR, S, D), lambda i: (i, 0, 0))
    out = pl.pallas_call(
        _copy_body,
        grid=(rows // BR,),
        in_specs=[spec, spec],
        out_specs=[spec, spec],
        out_shape=[jax.ShapeDtypeStruct((rows, S, D), k_val.dtype)] * 2,
    )(k2, v2)
    return out[0].reshape(B, H, S, D), out[1].reshape(B, H, S, D)


# speedup vs baseline: 48.7638x; 1.0154x over previous
import jax
import jax.numpy as jnp
from jax.experimental import pallas as pl

BR = 4


def _copy_body(k_ref, v_ref, ko_ref, vo_ref):
    ko_ref[...] = k_ref[...]
    vo_ref[...] = v_ref[...]


def kernel(k_val, v_val, k_cache, v_cache):
    B, H, S, D = k_val.shape
    rows = B * H
    k2 = k_val.reshape(rows, S, D)
    v2 = v_val.reshape(rows, S, D)
    spec = pl.BlockSpec((BR, S, D), lambda i: (i, 0, 0))
    out = pl.pallas_call(
        _copy_body,
        grid=(rows // BR,),
        in_specs=[spec, spec],
        out_specs=[spec, spec],
        out_shape=[jax.ShapeDtypeStruct((rows, S, D), k_val.dtype)] * 2,
    )(k2, v2)
    return out[0].reshape(B, H, S, D), out[1].reshape(B, H, S, D)
